# baseline (speedup 1.0000x reference)
import jax
import jax.numpy as jnp
from jax import lax
from jax.experimental import pallas as pl
from jax.experimental.pallas import tpu as pltpu
from jax.experimental.pallas import tpu_sc as plsc

B = 16384
L = 20
D_TITLE = 32
D_CAST = 16
D_KEY = 16
D_MOOD = 32
D_OUT = 96
V = 100000

_info = plsc.get_sparse_core_info()
NC, NS = _info.num_cores, _info.num_subcores
NW = NC * NS
BW = B // NW

CBLK = 512
NFULL = V // CBLK
TAIL = V - NFULL * CBLK
TAIL_PAD = 256
NITER = NFULL // NW + 1
TAIL_WID = 31

def _transpose_cols(ib, ob, d, scale, ncols):
  n_hi = d // 16
  iota16 = lax.broadcasted_iota(jnp.int32, (16,), 0)

  def col_body(it, _):
    c0 = it * 8
    for dc in range(8):
      c = c0 + dc
      idx_c = jnp.zeros((16,), jnp.int32) + c
      for h in range(n_hi):
        v = plsc.load_gather(ib, [iota16 + (16 * h), idx_c]) * scale
        ob[pl.ds(c * d + 16 * h, 16)] = v
    return 0

  lax.fori_loop(0, ncols // 8, col_body, 0)


def _transpose_blocks(tbl_hbm, tail_hbm, out_hbm, d, scale, in_bufs, out_bufs,
                      sem_in, sem_out, wid):
  def fire_in(b, par):
    pltpu.make_async_copy(
        tbl_hbm.at[:, pl.ds(b * CBLK, CBLK)], in_bufs[par], sem_in).start()

  def wait_in(b, par):
    pltpu.make_async_copy(
        tbl_hbm.at[:, pl.ds(b * CBLK, CBLK)], in_bufs[par], sem_in).wait()

  def fire_out(b, par):
    pltpu.make_async_copy(
        out_bufs[par],
        out_hbm.at[pl.ds(b * CBLK * d, CBLK * d)], sem_out).start()

  def wait_out(b, par):
    pltpu.make_async_copy(
        out_bufs[par],
        out_hbm.at[pl.ds(b * CBLK * d, CBLK * d)], sem_out).wait()

  def do(action, i):
    b = wid + NW * i
    par = i % 2
    pl.when(b < NFULL)(lambda: action(b, par))

  do(fire_in, 0)
  for i in range(NITER):
    if i + 1 < NITER:
      do(fire_in, i + 1)
    do(wait_in, i)
    if i >= 2:
      do(wait_out, i - 2)
    b = wid + NW * i
    par = i % 2
    pl.when(b < NFULL)(
        lambda: _transpose_cols(in_bufs[par], out_bufs[par], d, scale, CBLK))
    do(fire_out, i)
  for i in range(max(0, NITER - 2), NITER):
    do(wait_out, i)

  @pl.when(wid == TAIL_WID)
  def _():
    pltpu.sync_copy(tail_hbm, in_bufs[0].at[:, pl.ds(0, TAIL_PAD)])
    _transpose_cols(in_bufs[0], out_bufs[0], d, scale, TAIL)
    pltpu.sync_copy(out_bufs[0].at[pl.ds(0, TAIL * d)],
                    out_hbm.at[pl.ds(NFULL * CBLK * d, TAIL * d)])


def _detile_kernel(cT, kT, tT, c_tail, k_tail, t_tail,
                   cast_lin, key_lin, title_lin,
                   in16a, in16b, out16a, out16b,
                   in32a, in32b, out32a, out32b,
                   sem_in, sem_out):
  wid = lax.axis_index("s") * NC + lax.axis_index("c")
  inv_l = jnp.float32(1.0 / L)
  _transpose_blocks(cT, c_tail, cast_lin, D_CAST, inv_l, (in16a, in16b),
                    (out16a, out16b), sem_in, sem_out, wid)
  _transpose_blocks(kT, k_tail, key_lin, D_KEY, inv_l, (in16a, in16b),
                    (out16a, out16b), sem_in, sem_out, wid)
  _transpose_blocks(tT, t_tail, title_lin, D_TITLE, jnp.float32(1.0),
                    (in32a, in32b), (out32a, out32b), sem_in, sem_out, wid)


def _lookup_kernel(title_ids, castT, keyT, mood_ids,
                   title_u, cast_u, key_u, mood_table,
                   out_hbm,
                   tidx_v, midx_v, cidx_v, kidx_v,
                   t_rows, m_rows, acc_c, acc_k,
                   sem_i, sem_g, sem_o):
  wid = lax.axis_index("s") * NC + lax.axis_index("c")
  row0 = wid * BW

  idx_cps = [
      pltpu.make_async_copy(title_ids.at[pl.ds(row0, BW)], tidx_v, sem_i),
      pltpu.make_async_copy(mood_ids.at[pl.ds(row0, BW)], midx_v, sem_i),
      pltpu.make_async_copy(castT.at[:, pl.ds(row0, BW)], cidx_v, sem_i),
      pltpu.make_async_copy(keyT.at[:, pl.ds(row0, BW)], kidx_v, sem_i),
  ]
  for cp in idx_cps:
    cp.start()

  def zero_body(i, _):
    acc_c[i, :] = jnp.zeros((16,), jnp.float32)
    acc_k[i, :] = jnp.zeros((16,), jnp.float32)
    return 0
  lax.fori_loop(0, BW, zero_body, 0)

  for cp in idx_cps:
    cp.wait()

  g_cps = [
      pltpu.async_copy(title_u.at[tidx_v], t_rows, sem_g),
      pltpu.async_copy(mood_table.at[midx_v], m_rows, sem_g),
  ]
  for j in range(L):
    g_cps.append(
        pltpu.async_copy(cast_u.at[cidx_v.at[j]], acc_c, sem_g, add=True))
    g_cps.append(
        pltpu.async_copy(key_u.at[kidx_v.at[j]], acc_k, sem_g, add=True))
  for cp in g_cps:
    cp.wait()

  rows = pl.ds(row0, BW)
  out_cps = [
      pltpu.make_async_copy(t_rows, out_hbm.at[rows, pl.ds(0, D_TITLE)], sem_o),
      pltpu.make_async_copy(acc_c, out_hbm.at[rows, pl.ds(32, D_CAST)], sem_o),
      pltpu.make_async_copy(acc_k, out_hbm.at[rows, pl.ds(48, D_KEY)], sem_o),
      pltpu.make_async_copy(m_rows, out_hbm.at[rows, pl.ds(64, D_MOOD)], sem_o),
  ]
  for cp in out_cps:
    cp.start()
  for cp in out_cps:
    cp.wait()


@jax.jit
def _run(title_ids, castT, keyT, mood_ids,
         title_table, cast_table, key_table, mood_table):
  mesh = plsc.VectorSubcoreMesh(core_axis_name="c", subcore_axis_name="s")

  def tail_of(tbl):
    t = lax.slice_in_dim(tbl, NFULL * CBLK, V, axis=0)
    return jnp.pad(t, ((0, TAIL_PAD - TAIL), (0, 0))).T

  cast_lin, key_lin, title_lin = pl.kernel(
      _detile_kernel,
      mesh=mesh,
      compiler_params=pltpu.CompilerParams(
          use_tc_tiling_on_sc=True, needs_layout_passes=False),
      out_type=(
          jax.ShapeDtypeStruct((V * D_CAST,), jnp.float32),
          jax.ShapeDtypeStruct((V * D_KEY,), jnp.float32),
          jax.ShapeDtypeStruct((V * D_TITLE,), jnp.float32),
      ),
      scratch_types=[
          pltpu.VMEM((16, CBLK), jnp.float32),
          pltpu.VMEM((16, CBLK), jnp.float32),
          pltpu.VMEM((CBLK * 16,), jnp.float32),
          pltpu.VMEM((CBLK * 16,), jnp.float32),
          pltpu.VMEM((32, CBLK), jnp.float32),
          pltpu.VMEM((32, CBLK), jnp.float32),
          pltpu.VMEM((CBLK * 32,), jnp.float32),
          pltpu.VMEM((CBLK * 32,), jnp.float32),
          pltpu.SemaphoreType.DMA,
          pltpu.SemaphoreType.DMA,
      ],
  )(cast_table.T, key_table.T, title_table.T,
    tail_of(cast_table), tail_of(key_table), tail_of(title_table))

  cast_u = cast_lin.reshape(V, D_CAST)
  key_u = key_lin.reshape(V, D_KEY)
  title_u = title_lin.reshape(V, D_TITLE)

  return pl.kernel(
      _lookup_kernel,
      mesh=mesh,
      compiler_params=pltpu.CompilerParams(use_tc_tiling_on_sc=False),
      out_type=jax.ShapeDtypeStruct((B, D_OUT), jnp.float32),
      scratch_types=[
          pltpu.VMEM((BW,), jnp.int32),
          pltpu.VMEM((BW,), jnp.int32),
          pltpu.VMEM((L, BW), jnp.int32),
          pltpu.VMEM((L, BW), jnp.int32),
          pltpu.VMEM((BW, D_TITLE), jnp.float32),
          pltpu.VMEM((BW, D_MOOD), jnp.float32),
          pltpu.VMEM((BW, D_CAST), jnp.float32),
          pltpu.VMEM((BW, D_KEY), jnp.float32),
          pltpu.SemaphoreType.DMA,
          pltpu.SemaphoreType.DMA,
          pltpu.SemaphoreType.DMA,
      ],
  )(title_ids, castT, keyT, mood_ids,
    title_u, cast_u, key_u, mood_table)


def kernel(title_ids, cast_ids, keyword_ids, mood_ids,
           title_table, cast_table, key_table, mood_table):
  return _run(title_ids, cast_ids.T, keyword_ids.T, mood_ids,
              title_table, cast_table, key_table, mood_table)


# speedup vs baseline: 1.4096x; 1.4096x over previous
import jax
import jax.numpy as jnp
from jax import lax
from jax.experimental import pallas as pl
from jax.experimental.pallas import tpu as pltpu
from jax.experimental.pallas import tpu_sc as plsc

B = 16384
L = 20
D_TITLE = 32
D_CAST = 16
D_KEY = 16
D_MOOD = 32
D_OUT = 96
V = 100000

_info = plsc.get_sparse_core_info()
NC, NS = _info.num_cores, _info.num_subcores
NW = NC * NS
BW = B // NW

CBLK = 512
NFULL = V // CBLK
TAIL = V - NFULL * CBLK
TAIL_PAD = 256
NITER = NFULL // NW + 1
TAIL_WID = 31

def _transpose_cols(ib, ob, d, scale, ncols):
  iota16 = lax.broadcasted_iota(jnp.int32, (16,), 0)

  def chunk_body(it, _):
    c0 = it * 16
    base_idx = (iota16 + c0) * d
    for dd in range(d):
      v = ib[dd, pl.ds(c0, 16)] * scale
      plsc.store_scatter(ob, [base_idx + dd], v)
    return 0

  lax.fori_loop(0, ncols // 16, chunk_body, 0)


def _transpose_blocks(tbl_hbm, tail_hbm, out_hbm, d, scale, in_bufs, out_bufs,
                      sem_in, sem_out, wid):
  def fire_in(b, par):
    pltpu.make_async_copy(
        tbl_hbm.at[:, pl.ds(b * CBLK, CBLK)], in_bufs[par], sem_in).start()

  def wait_in(b, par):
    pltpu.make_async_copy(
        tbl_hbm.at[:, pl.ds(b * CBLK, CBLK)], in_bufs[par], sem_in).wait()

  def fire_out(b, par):
    pltpu.make_async_copy(
        out_bufs[par],
        out_hbm.at[pl.ds(b * CBLK * d, CBLK * d)], sem_out).start()

  def wait_out(b, par):
    pltpu.make_async_copy(
        out_bufs[par],
        out_hbm.at[pl.ds(b * CBLK * d, CBLK * d)], sem_out).wait()

  def do(action, i):
    b = wid + NW * i
    par = i % 2
    pl.when(b < NFULL)(lambda: action(b, par))

  do(fire_in, 0)
  for i in range(NITER):
    if i + 1 < NITER:
      do(fire_in, i + 1)
    do(wait_in, i)
    if i >= 2:
      do(wait_out, i - 2)
    b = wid + NW * i
    par = i % 2
    pl.when(b < NFULL)(
        lambda: _transpose_cols(in_bufs[par], out_bufs[par], d, scale, CBLK))
    do(fire_out, i)
  for i in range(max(0, NITER - 2), NITER):
    do(wait_out, i)

  @pl.when(wid == TAIL_WID)
  def _():
    pltpu.sync_copy(tail_hbm, in_bufs[0].at[:, pl.ds(0, TAIL_PAD)])
    _transpose_cols(in_bufs[0], out_bufs[0], d, scale, TAIL)
    pltpu.sync_copy(out_bufs[0].at[pl.ds(0, TAIL * d)],
                    out_hbm.at[pl.ds(NFULL * CBLK * d, TAIL * d)])


def _detile_kernel(cT, kT, tT, c_tail, k_tail, t_tail,
                   cast_lin, key_lin, title_lin,
                   in16a, in16b, out16a, out16b,
                   in32a, in32b, out32a, out32b,
                   sem_in, sem_out):
  wid = lax.axis_index("s") * NC + lax.axis_index("c")
  inv_l = jnp.float32(1.0 / L)
  _transpose_blocks(cT, c_tail, cast_lin, D_CAST, inv_l, (in16a, in16b),
                    (out16a, out16b), sem_in, sem_out, wid)
  _transpose_blocks(kT, k_tail, key_lin, D_KEY, inv_l, (in16a, in16b),
                    (out16a, out16b), sem_in, sem_out, wid)
  _transpose_blocks(tT, t_tail, title_lin, D_TITLE, jnp.float32(1.0),
                    (in32a, in32b), (out32a, out32b), sem_in, sem_out, wid)


def _lookup_kernel(title_ids, castT, keyT, mood_ids,
                   title_u, cast_u, key_u, mood_table,
                   out_hbm,
                   tidx_v, midx_v, cidx_v, kidx_v,
                   t_rows, m_rows, acc_c, acc_k,
                   sem_i, sem_g, sem_o):
  wid = lax.axis_index("s") * NC + lax.axis_index("c")
  row0 = wid * BW

  idx_cps = [
      pltpu.make_async_copy(title_ids.at[pl.ds(row0, BW)], tidx_v, sem_i),
      pltpu.make_async_copy(mood_ids.at[pl.ds(row0, BW)], midx_v, sem_i),
      pltpu.make_async_copy(castT.at[:, pl.ds(row0, BW)], cidx_v, sem_i),
      pltpu.make_async_copy(keyT.at[:, pl.ds(row0, BW)], kidx_v, sem_i),
  ]
  for cp in idx_cps:
    cp.start()

  def zero_body(i, _):
    acc_c[i, :] = jnp.zeros((16,), jnp.float32)
    acc_k[i, :] = jnp.zeros((16,), jnp.float32)
    return 0
  lax.fori_loop(0, BW, zero_body, 0)

  for cp in idx_cps:
    cp.wait()

  g_cps = [
      pltpu.async_copy(title_u.at[tidx_v], t_rows, sem_g),
      pltpu.async_copy(mood_table.at[midx_v], m_rows, sem_g),
  ]
  for j in range(L):
    g_cps.append(
        pltpu.async_copy(cast_u.at[cidx_v.at[j]], acc_c, sem_g, add=True))
    g_cps.append(
        pltpu.async_copy(key_u.at[kidx_v.at[j]], acc_k, sem_g, add=True))
  for cp in g_cps:
    cp.wait()

  rows = pl.ds(row0, BW)
  out_cps = [
      pltpu.make_async_copy(t_rows, out_hbm.at[rows, pl.ds(0, D_TITLE)], sem_o),
      pltpu.make_async_copy(acc_c, out_hbm.at[rows, pl.ds(32, D_CAST)], sem_o),
      pltpu.make_async_copy(acc_k, out_hbm.at[rows, pl.ds(48, D_KEY)], sem_o),
      pltpu.make_async_copy(m_rows, out_hbm.at[rows, pl.ds(64, D_MOOD)], sem_o),
  ]
  for cp in out_cps:
    cp.start()
  for cp in out_cps:
    cp.wait()


@jax.jit
def _run(title_ids, castT, keyT, mood_ids,
         title_table, cast_table, key_table, mood_table):
  mesh = plsc.VectorSubcoreMesh(core_axis_name="c", subcore_axis_name="s")

  def tail_of(tbl):
    t = lax.slice_in_dim(tbl, NFULL * CBLK, V, axis=0)
    return jnp.pad(t, ((0, TAIL_PAD - TAIL), (0, 0))).T

  cast_lin, key_lin, title_lin = pl.kernel(
      _detile_kernel,
      mesh=mesh,
      compiler_params=pltpu.CompilerParams(
          use_tc_tiling_on_sc=True, needs_layout_passes=False),
      out_type=(
          jax.ShapeDtypeStruct((V * D_CAST,), jnp.float32),
          jax.ShapeDtypeStruct((V * D_KEY,), jnp.float32),
          jax.ShapeDtypeStruct((V * D_TITLE,), jnp.float32),
      ),
      scratch_types=[
          pltpu.VMEM((16, CBLK), jnp.float32),
          pltpu.VMEM((16, CBLK), jnp.float32),
          pltpu.VMEM((CBLK * 16,), jnp.float32),
          pltpu.VMEM((CBLK * 16,), jnp.float32),
          pltpu.VMEM((32, CBLK), jnp.float32),
          pltpu.VMEM((32, CBLK), jnp.float32),
          pltpu.VMEM((CBLK * 32,), jnp.float32),
          pltpu.VMEM((CBLK * 32,), jnp.float32),
          pltpu.SemaphoreType.DMA,
          pltpu.SemaphoreType.DMA,
      ],
  )(cast_table.T, key_table.T, title_table.T,
    tail_of(cast_table), tail_of(key_table), tail_of(title_table))

  cast_u = cast_lin.reshape(V, D_CAST)
  key_u = key_lin.reshape(V, D_KEY)
  title_u = title_lin.reshape(V, D_TITLE)

  return pl.kernel(
      _lookup_kernel,
      mesh=mesh,
      compiler_params=pltpu.CompilerParams(use_tc_tiling_on_sc=False),
      out_type=jax.ShapeDtypeStruct((B, D_OUT), jnp.float32),
      scratch_types=[
          pltpu.VMEM((BW,), jnp.int32),
          pltpu.VMEM((BW,), jnp.int32),
          pltpu.VMEM((L, BW), jnp.int32),
          pltpu.VMEM((L, BW), jnp.int32),
          pltpu.VMEM((BW, D_TITLE), jnp.float32),
          pltpu.VMEM((BW, D_MOOD), jnp.float32),
          pltpu.VMEM((BW, D_CAST), jnp.float32),
          pltpu.VMEM((BW, D_KEY), jnp.float32),
          pltpu.SemaphoreType.DMA,
          pltpu.SemaphoreType.DMA,
          pltpu.SemaphoreType.DMA,
      ],
  )(title_ids, castT, keyT, mood_ids,
    title_u, cast_u, key_u, mood_table)


def kernel(title_ids, cast_ids, keyword_ids, mood_ids,
           title_table, cast_table, key_table, mood_table):
  return _run(title_ids, cast_ids.T, keyword_ids.T, mood_ids,
              title_table, cast_table, key_table, mood_table)


# speedup vs baseline: 1.4807x; 1.0504x over previous
import jax
import jax.numpy as jnp
from jax import lax
from jax.experimental import pallas as pl
from jax.experimental.pallas import tpu as pltpu
from jax.experimental.pallas import tpu_sc as plsc

B = 16384
L = 20
D_TITLE = 32
D_CAST = 16
D_KEY = 16
D_MOOD = 32
D_OUT = 96
V = 100000

_info = plsc.get_sparse_core_info()
NC, NS = _info.num_cores, _info.num_subcores
NW = NC * NS
BW = B // NW

CBLK = 512
NFULL = V // CBLK
TAIL = V - NFULL * CBLK
TAIL_PAD = 256
NITER = NFULL // NW + 1
TAIL_WID = 31

def _transpose_cols(ib, ob, d, scale, ncols):
  iota16 = lax.broadcasted_iota(jnp.int32, (16,), 0)

  def chunk_body(it, _):
    c0 = it * 16
    base_idx = (iota16 + c0) * d
    for dd in range(d):
      v = ib[dd, pl.ds(c0, 16)]
      if scale is not None:
        v = v * scale
      plsc.store_scatter(ob, [base_idx + dd], v)
    return 0

  lax.fori_loop(0, ncols // 16, chunk_body, 0)


def _transpose_blocks(tbl_hbm, tail_hbm, out_hbm, d, scale, in_bufs, out_bufs,
                      sem_in, sem_out, wid):
  def fire_in(b, par):
    pltpu.make_async_copy(
        tbl_hbm.at[:, pl.ds(b * CBLK, CBLK)], in_bufs[par], sem_in).start()

  def wait_in(b, par):
    pltpu.make_async_copy(
        tbl_hbm.at[:, pl.ds(b * CBLK, CBLK)], in_bufs[par], sem_in).wait()

  def fire_out(b, par):
    pltpu.make_async_copy(
        out_bufs[par],
        out_hbm.at[pl.ds(b * CBLK * d, CBLK * d)], sem_out).start()

  def wait_out(b, par):
    pltpu.make_async_copy(
        out_bufs[par],
        out_hbm.at[pl.ds(b * CBLK * d, CBLK * d)], sem_out).wait()

  def do(action, i):
    b = wid + NW * i
    par = i % 2
    pl.when(b < NFULL)(lambda: action(b, par))

  do(fire_in, 0)
  for i in range(NITER):
    if i + 1 < NITER:
      do(fire_in, i + 1)
    do(wait_in, i)
    if i >= 2:
      do(wait_out, i - 2)
    b = wid + NW * i
    par = i % 2
    pl.when(b < NFULL)(
        lambda: _transpose_cols(in_bufs[par], out_bufs[par], d, scale, CBLK))
    do(fire_out, i)
  for i in range(max(0, NITER - 2), NITER):
    do(wait_out, i)

  @pl.when(wid == TAIL_WID)
  def _():
    pltpu.sync_copy(tail_hbm, in_bufs[0].at[:, pl.ds(0, TAIL_PAD)])
    _transpose_cols(in_bufs[0], out_bufs[0], d, scale, TAIL)
    pltpu.sync_copy(out_bufs[0].at[pl.ds(0, TAIL * d)],
                    out_hbm.at[pl.ds(NFULL * CBLK * d, TAIL * d)])


def _detile_kernel(cT, kT, tT, c_tail, k_tail, t_tail,
                   cast_lin, key_lin, title_lin,
                   in16a, in16b, out16a, out16b,
                   in32a, in32b, out32a, out32b,
                   sem_in, sem_out):
  wid = lax.axis_index("s") * NC + lax.axis_index("c")
  _transpose_blocks(cT, c_tail, cast_lin, D_CAST, None, (in16a, in16b),
                    (out16a, out16b), sem_in, sem_out, wid)
  _transpose_blocks(kT, k_tail, key_lin, D_KEY, None, (in16a, in16b),
                    (out16a, out16b), sem_in, sem_out, wid)
  _transpose_blocks(tT, t_tail, title_lin, D_TITLE, None,
                    (in32a, in32b), (out32a, out32b), sem_in, sem_out, wid)


def _lookup_kernel(title_ids, castT, keyT, mood_ids,
                   title_u, cast_u, key_u, mood_table,
                   out_hbm,
                   tidx_v, midx_v, cidx_v, kidx_v,
                   t_rows, m_rows, acc_c, acc_k,
                   sem_i, sem_g, sem_o):
  wid = lax.axis_index("s") * NC + lax.axis_index("c")
  row0 = wid * BW

  idx_cps = [
      pltpu.make_async_copy(title_ids.at[pl.ds(row0, BW)], tidx_v, sem_i),
      pltpu.make_async_copy(mood_ids.at[pl.ds(row0, BW)], midx_v, sem_i),
      pltpu.make_async_copy(castT.at[:, pl.ds(row0, BW)], cidx_v, sem_i),
      pltpu.make_async_copy(keyT.at[:, pl.ds(row0, BW)], kidx_v, sem_i),
  ]
  for cp in idx_cps:
    cp.start()

  def zero_body(i, _):
    acc_c[i, :] = jnp.zeros((16,), jnp.float32)
    acc_k[i, :] = jnp.zeros((16,), jnp.float32)
    return 0
  lax.fori_loop(0, BW, zero_body, 0)

  for cp in idx_cps:
    cp.wait()

  g_cps = [
      pltpu.async_copy(title_u.at[tidx_v], t_rows, sem_g),
      pltpu.async_copy(mood_table.at[midx_v], m_rows, sem_g),
  ]
  for j in range(L):
    g_cps.append(
        pltpu.async_copy(cast_u.at[cidx_v.at[j]], acc_c, sem_g, add=True))
    g_cps.append(
        pltpu.async_copy(key_u.at[kidx_v.at[j]], acc_k, sem_g, add=True))
  for cp in g_cps:
    cp.wait()

  inv_l = jnp.float32(1.0 / L)
  def scale_body(i, _):
    acc_c[i, :] = acc_c[i, :] * inv_l
    acc_k[i, :] = acc_k[i, :] * inv_l
    return 0
  lax.fori_loop(0, BW, scale_body, 0)

  rows = pl.ds(row0, BW)
  out_cps = [
      pltpu.make_async_copy(t_rows, out_hbm.at[rows, pl.ds(0, D_TITLE)], sem_o),
      pltpu.make_async_copy(acc_c, out_hbm.at[rows, pl.ds(32, D_CAST)], sem_o),
      pltpu.make_async_copy(acc_k, out_hbm.at[rows, pl.ds(48, D_KEY)], sem_o),
      pltpu.make_async_copy(m_rows, out_hbm.at[rows, pl.ds(64, D_MOOD)], sem_o),
  ]
  for cp in out_cps:
    cp.start()
  for cp in out_cps:
    cp.wait()


@jax.jit
def _run(title_ids, castT, keyT, mood_ids,
         title_table, cast_table, key_table, mood_table):
  mesh = plsc.VectorSubcoreMesh(core_axis_name="c", subcore_axis_name="s")

  def tail_of(tbl):
    t = lax.slice_in_dim(tbl, NFULL * CBLK, V, axis=0)
    return jnp.pad(t, ((0, TAIL_PAD - TAIL), (0, 0))).T

  cast_lin, key_lin, title_lin = pl.kernel(
      _detile_kernel,
      mesh=mesh,
      compiler_params=pltpu.CompilerParams(
          use_tc_tiling_on_sc=True, needs_layout_passes=False),
      out_type=(
          jax.ShapeDtypeStruct((V * D_CAST,), jnp.float32),
          jax.ShapeDtypeStruct((V * D_KEY,), jnp.float32),
          jax.ShapeDtypeStruct((V * D_TITLE,), jnp.float32),
      ),
      scratch_types=[
          pltpu.VMEM((16, CBLK), jnp.float32),
          pltpu.VMEM((16, CBLK), jnp.float32),
          pltpu.VMEM((CBLK * 16,), jnp.float32),
          pltpu.VMEM((CBLK * 16,), jnp.float32),
          pltpu.VMEM((32, CBLK), jnp.float32),
          pltpu.VMEM((32, CBLK), jnp.float32),
          pltpu.VMEM((CBLK * 32,), jnp.float32),
          pltpu.VMEM((CBLK * 32,), jnp.float32),
          pltpu.SemaphoreType.DMA,
          pltpu.SemaphoreType.DMA,
      ],
  )(cast_table.T, key_table.T, title_table.T,
    tail_of(cast_table), tail_of(key_table), tail_of(title_table))

  cast_u = cast_lin.reshape(V, D_CAST)
  key_u = key_lin.reshape(V, D_KEY)
  title_u = title_lin.reshape(V, D_TITLE)

  return pl.kernel(
      _lookup_kernel,
      mesh=mesh,
      compiler_params=pltpu.CompilerParams(use_tc_tiling_on_sc=False),
      out_type=jax.ShapeDtypeStruct((B, D_OUT), jnp.float32),
      scratch_types=[
          pltpu.VMEM((BW,), jnp.int32),
          pltpu.VMEM((BW,), jnp.int32),
          pltpu.VMEM((L, BW), jnp.int32),
          pltpu.VMEM((L, BW), jnp.int32),
          pltpu.VMEM((BW, D_TITLE), jnp.float32),
          pltpu.VMEM((BW, D_MOOD), jnp.float32),
          pltpu.VMEM((BW, D_CAST), jnp.float32),
          pltpu.VMEM((BW, D_KEY), jnp.float32),
          pltpu.SemaphoreType.DMA,
          pltpu.SemaphoreType.DMA,
          pltpu.SemaphoreType.DMA,
      ],
  )(title_ids, castT, keyT, mood_ids,
    title_u, cast_u, key_u, mood_table)


def kernel(title_ids, cast_ids, keyword_ids, mood_ids,
           title_table, cast_table, key_table, mood_table):
  return _run(title_ids, cast_ids.T, keyword_ids.T, mood_ids,
              title_table, cast_table, key_table, mood_table)


# speedup vs baseline: 1.6617x; 1.1222x over previous
import jax
import jax.numpy as jnp
from jax import lax
from jax.experimental import pallas as pl
from jax.experimental.pallas import tpu as pltpu
from jax.experimental.pallas import tpu_sc as plsc

B = 16384
L = 20
D_TITLE = 32
D_CAST = 16
D_KEY = 16
D_MOOD = 32
D_OUT = 96
V = 100000

_info = plsc.get_sparse_core_info()
NC, NS = _info.num_cores, _info.num_subcores
NW = NC * NS
BW = B // NW

CBLK = 512
NFULL = V // CBLK
TAIL = V - NFULL * CBLK
TAIL_PAD = 256
NITER = NFULL // NW + 1
TAIL_WID = 31

def _transpose_cols(ib, ob, d, scale, ncols):
  iota16 = lax.broadcasted_iota(jnp.int32, (16,), 0)

  @plsc.parallel_loop(0, ncols // 16, unroll=2)
  def chunk_body(it):
    c0 = it * 16
    base_idx = (iota16 + c0) * d
    for dd in range(d):
      v = ib[dd, pl.ds(c0, 16)]
      if scale is not None:
        v = v * scale
      plsc.store_scatter(ob, [base_idx + dd], v)


def _transpose_blocks(tbl_hbm, tail_hbm, out_hbm, d, scale, in_bufs, out_bufs,
                      sem_in, sem_out, wid):
  def fire_in(b, par):
    pltpu.make_async_copy(
        tbl_hbm.at[:, pl.ds(b * CBLK, CBLK)], in_bufs[par], sem_in).start()

  def wait_in(b, par):
    pltpu.make_async_copy(
        tbl_hbm.at[:, pl.ds(b * CBLK, CBLK)], in_bufs[par], sem_in).wait()

  def fire_out(b, par):
    pltpu.make_async_copy(
        out_bufs[par],
        out_hbm.at[pl.ds(b * CBLK * d, CBLK * d)], sem_out).start()

  def wait_out(b, par):
    pltpu.make_async_copy(
        out_bufs[par],
        out_hbm.at[pl.ds(b * CBLK * d, CBLK * d)], sem_out).wait()

  def do(action, i):
    b = wid + NW * i
    par = i % 2
    pl.when(b < NFULL)(lambda: action(b, par))

  do(fire_in, 0)
  for i in range(NITER):
    if i + 1 < NITER:
      do(fire_in, i + 1)
    do(wait_in, i)
    if i >= 2:
      do(wait_out, i - 2)
    b = wid + NW * i
    par = i % 2
    pl.when(b < NFULL)(
        lambda: _transpose_cols(in_bufs[par], out_bufs[par], d, scale, CBLK))
    do(fire_out, i)
  for i in range(max(0, NITER - 2), NITER):
    do(wait_out, i)

  @pl.when(wid == TAIL_WID)
  def _():
    pltpu.sync_copy(tail_hbm, in_bufs[0].at[:, pl.ds(0, TAIL_PAD)])
    _transpose_cols(in_bufs[0], out_bufs[0], d, scale, TAIL)
    pltpu.sync_copy(out_bufs[0].at[pl.ds(0, TAIL * d)],
                    out_hbm.at[pl.ds(NFULL * CBLK * d, TAIL * d)])


def _detile_kernel(cT, kT, tT, c_tail, k_tail, t_tail,
                   cast_lin, key_lin, title_lin,
                   in16a, in16b, out16a, out16b,
                   in32a, in32b, out32a, out32b,
                   sem_in, sem_out):
  wid = lax.axis_index("s") * NC + lax.axis_index("c")
  _transpose_blocks(cT, c_tail, cast_lin, D_CAST, None, (in16a, in16b),
                    (out16a, out16b), sem_in, sem_out, wid)
  _transpose_blocks(kT, k_tail, key_lin, D_KEY, None, (in16a, in16b),
                    (out16a, out16b), sem_in, sem_out, wid)
  _transpose_blocks(tT, t_tail, title_lin, D_TITLE, None,
                    (in32a, in32b), (out32a, out32b), sem_in, sem_out, wid)


def _lookup_kernel(title_ids, castT, keyT, mood_ids,
                   title_u, cast_u, key_u, mood_table,
                   out_hbm,
                   tidx_v, midx_v, cidx_v, kidx_v,
                   t_rows, m_rows, acc_c, acc_k,
                   sem_i, sem_g, sem_o):
  wid = lax.axis_index("s") * NC + lax.axis_index("c")
  row0 = wid * BW

  idx_cps = [
      pltpu.make_async_copy(title_ids.at[pl.ds(row0, BW)], tidx_v, sem_i),
      pltpu.make_async_copy(mood_ids.at[pl.ds(row0, BW)], midx_v, sem_i),
      pltpu.make_async_copy(castT.at[:, pl.ds(row0, BW)], cidx_v, sem_i),
      pltpu.make_async_copy(keyT.at[:, pl.ds(row0, BW)], kidx_v, sem_i),
  ]
  for cp in idx_cps:
    cp.start()

  @plsc.parallel_loop(0, BW, unroll=4)
  def zero_body(i):
    acc_c[i, :] = jnp.zeros((16,), jnp.float32)
    acc_k[i, :] = jnp.zeros((16,), jnp.float32)

  for cp in idx_cps:
    cp.wait()

  g_cps = [
      pltpu.async_copy(title_u.at[tidx_v], t_rows, sem_g),
      pltpu.async_copy(mood_table.at[midx_v], m_rows, sem_g),
  ]
  for j in range(L):
    g_cps.append(
        pltpu.async_copy(cast_u.at[cidx_v.at[j]], acc_c, sem_g, add=True))
    g_cps.append(
        pltpu.async_copy(key_u.at[kidx_v.at[j]], acc_k, sem_g, add=True))
  for cp in g_cps:
    cp.wait()

  inv_l = jnp.float32(1.0 / L)

  @plsc.parallel_loop(0, BW, unroll=4)
  def scale_body(i):
    acc_c[i, :] = acc_c[i, :] * inv_l
    acc_k[i, :] = acc_k[i, :] * inv_l

  rows = pl.ds(row0, BW)
  out_cps = [
      pltpu.make_async_copy(t_rows, out_hbm.at[rows, pl.ds(0, D_TITLE)], sem_o),
      pltpu.make_async_copy(acc_c, out_hbm.at[rows, pl.ds(32, D_CAST)], sem_o),
      pltpu.make_async_copy(acc_k, out_hbm.at[rows, pl.ds(48, D_KEY)], sem_o),
      pltpu.make_async_copy(m_rows, out_hbm.at[rows, pl.ds(64, D_MOOD)], sem_o),
  ]
  for cp in out_cps:
    cp.start()
  for cp in out_cps:
    cp.wait()


@jax.jit
def _run(title_ids, castT, keyT, mood_ids,
         title_table, cast_table, key_table, mood_table):
  mesh = plsc.VectorSubcoreMesh(core_axis_name="c", subcore_axis_name="s")

  def tail_of(tbl):
    t = lax.slice_in_dim(tbl, NFULL * CBLK, V, axis=0)
    return jnp.pad(t, ((0, TAIL_PAD - TAIL), (0, 0))).T

  cast_lin, key_lin, title_lin = pl.kernel(
      _detile_kernel,
      mesh=mesh,
      compiler_params=pltpu.CompilerParams(
          use_tc_tiling_on_sc=True, needs_layout_passes=False),
      out_type=(
          jax.ShapeDtypeStruct((V * D_CAST,), jnp.float32),
          jax.ShapeDtypeStruct((V * D_KEY,), jnp.float32),
          jax.ShapeDtypeStruct((V * D_TITLE,), jnp.float32),
      ),
      scratch_types=[
          pltpu.VMEM((16, CBLK), jnp.float32),
          pltpu.VMEM((16, CBLK), jnp.float32),
          pltpu.VMEM((CBLK * 16,), jnp.float32),
          pltpu.VMEM((CBLK * 16,), jnp.float32),
          pltpu.VMEM((32, CBLK), jnp.float32),
          pltpu.VMEM((32, CBLK), jnp.float32),
          pltpu.VMEM((CBLK * 32,), jnp.float32),
          pltpu.VMEM((CBLK * 32,), jnp.float32),
          pltpu.SemaphoreType.DMA,
          pltpu.SemaphoreType.DMA,
      ],
  )(cast_table.T, key_table.T, title_table.T,
    tail_of(cast_table), tail_of(key_table), tail_of(title_table))

  cast_u = cast_lin.reshape(V, D_CAST)
  key_u = key_lin.reshape(V, D_KEY)
  title_u = title_lin.reshape(V, D_TITLE)

  return pl.kernel(
      _lookup_kernel,
      mesh=mesh,
      compiler_params=pltpu.CompilerParams(use_tc_tiling_on_sc=False),
      out_type=jax.ShapeDtypeStruct((B, D_OUT), jnp.float32),
      scratch_types=[
          pltpu.VMEM((BW,), jnp.int32),
          pltpu.VMEM((BW,), jnp.int32),
          pltpu.VMEM((L, BW), jnp.int32),
          pltpu.VMEM((L, BW), jnp.int32),
          pltpu.VMEM((BW, D_TITLE), jnp.float32),
          pltpu.VMEM((BW, D_MOOD), jnp.float32),
          pltpu.VMEM((BW, D_CAST), jnp.float32),
          pltpu.VMEM((BW, D_KEY), jnp.float32),
          pltpu.SemaphoreType.DMA,
          pltpu.SemaphoreType.DMA,
          pltpu.SemaphoreType.DMA,
      ],
  )(title_ids, castT, keyT, mood_ids,
    title_u, cast_u, key_u, mood_table)


def kernel(title_ids, cast_ids, keyword_ids, mood_ids,
           title_table, cast_table, key_table, mood_table):
  return _run(title_ids, cast_ids.T, keyword_ids.T, mood_ids,
              title_table, cast_table, key_table, mood_table)


# speedup vs baseline: 1.6715x; 1.0059x over previous
import jax
import jax.numpy as jnp
from jax import lax
from jax.experimental import pallas as pl
from jax.experimental.pallas import tpu as pltpu
from jax.experimental.pallas import tpu_sc as plsc

B = 16384
L = 20
D_TITLE = 32
D_CAST = 16
D_KEY = 16
D_MOOD = 32
D_OUT = 96
V = 100000

_info = plsc.get_sparse_core_info()
NC, NS = _info.num_cores, _info.num_subcores
NW = NC * NS
BW = B // NW

CBLK = 512
NFULL = V // CBLK
TAIL = V - NFULL * CBLK
TAIL_PAD = 256
NITER = NFULL // NW + 1
TAIL_WID = 31

def _transpose_cols(ib, ob, d, scale, ncols):
  iota16 = lax.broadcasted_iota(jnp.int32, (16,), 0)

  @plsc.parallel_loop(0, ncols // 16, unroll=4)
  def chunk_body(it):
    c0 = it * 16
    base_idx = (iota16 + c0) * d
    for dd in range(d):
      v = ib[dd, pl.ds(c0, 16)]
      if scale is not None:
        v = v * scale
      plsc.store_scatter(ob, [base_idx + dd], v)


def _transpose_blocks(tbl_hbm, tail_hbm, out_hbm, d, scale, in_bufs, out_bufs,
                      sem_in, sem_out, wid):
  def fire_in(b, par):
    pltpu.make_async_copy(
        tbl_hbm.at[:, pl.ds(b * CBLK, CBLK)], in_bufs[par], sem_in).start()

  def wait_in(b, par):
    pltpu.make_async_copy(
        tbl_hbm.at[:, pl.ds(b * CBLK, CBLK)], in_bufs[par], sem_in).wait()

  def fire_out(b, par):
    pltpu.make_async_copy(
        out_bufs[par],
        out_hbm.at[pl.ds(b * CBLK * d, CBLK * d)], sem_out).start()

  def wait_out(b, par):
    pltpu.make_async_copy(
        out_bufs[par],
        out_hbm.at[pl.ds(b * CBLK * d, CBLK * d)], sem_out).wait()

  fire_in(wid, 0)

  def pair_body(i2, _):
    for par in (0, 1):
      b = wid + NW * (i2 * 2 + par)
      pl.when(b + NW < NFULL)(lambda: fire_in(b + NW, 1 - par))
      pl.when(b < NFULL)(lambda: wait_in(b, par))
      pl.when(jnp.logical_and(b >= 2 * NW, b - 2 * NW < NFULL))(
          lambda: wait_out(b - 2 * NW, par))
      pl.when(b < NFULL)(
          lambda: _transpose_cols(in_bufs[par], out_bufs[par], d, scale, CBLK))
      pl.when(b < NFULL)(lambda: fire_out(b, par))
    return 0

  lax.fori_loop(0, (NITER + 1) // 2, pair_body, 0)

  b_last = wid + NW * (NITER - 1)
  pl.when(b_last < NFULL)(lambda: wait_out(b_last, (NITER - 1) % 2))

  @pl.when(wid == TAIL_WID)
  def _():
    pltpu.sync_copy(tail_hbm, in_bufs[0].at[:, pl.ds(0, TAIL_PAD)])
    _transpose_cols(in_bufs[0], out_bufs[0], d, scale, TAIL)
    pltpu.sync_copy(out_bufs[0].at[pl.ds(0, TAIL * d)],
                    out_hbm.at[pl.ds(NFULL * CBLK * d, TAIL * d)])


def _detile_kernel(cT, kT, tT, c_tail, k_tail, t_tail,
                   cast_lin, key_lin, title_lin,
                   in16a, in16b, out16a, out16b,
                   in32a, in32b, out32a, out32b,
                   sem_in, sem_out):
  wid = lax.axis_index("s") * NC + lax.axis_index("c")
  _transpose_blocks(cT, c_tail, cast_lin, D_CAST, None, (in16a, in16b),
                    (out16a, out16b), sem_in, sem_out, wid)
  _transpose_blocks(kT, k_tail, key_lin, D_KEY, None, (in16a, in16b),
                    (out16a, out16b), sem_in, sem_out, wid)
  _transpose_blocks(tT, t_tail, title_lin, D_TITLE, None,
                    (in32a, in32b), (out32a, out32b), sem_in, sem_out, wid)


def _lookup_kernel(title_ids, castT, keyT, mood_ids,
                   title_u, cast_u, key_u, mood_table,
                   out_hbm,
                   tidx_v, midx_v, cidx_v, kidx_v,
                   t_rows, m_rows, acc_c, acc_k,
                   sem_i, sem_g, sem_o):
  wid = lax.axis_index("s") * NC + lax.axis_index("c")
  row0 = wid * BW

  idx_cps = [
      pltpu.make_async_copy(title_ids.at[pl.ds(row0, BW)], tidx_v, sem_i),
      pltpu.make_async_copy(mood_ids.at[pl.ds(row0, BW)], midx_v, sem_i),
      pltpu.make_async_copy(castT.at[:, pl.ds(row0, BW)], cidx_v, sem_i),
      pltpu.make_async_copy(keyT.at[:, pl.ds(row0, BW)], kidx_v, sem_i),
  ]
  for cp in idx_cps:
    cp.start()

  @plsc.parallel_loop(0, BW, unroll=4)
  def zero_body(i):
    acc_c[i, :] = jnp.zeros((16,), jnp.float32)
    acc_k[i, :] = jnp.zeros((16,), jnp.float32)

  for cp in idx_cps:
    cp.wait()

  g_cps = [
      pltpu.async_copy(title_u.at[tidx_v], t_rows, sem_g),
      pltpu.async_copy(mood_table.at[midx_v], m_rows, sem_g),
  ]
  for j in range(L):
    g_cps.append(
        pltpu.async_copy(cast_u.at[cidx_v.at[j]], acc_c, sem_g, add=True))
    g_cps.append(
        pltpu.async_copy(key_u.at[kidx_v.at[j]], acc_k, sem_g, add=True))
  for cp in g_cps:
    cp.wait()

  inv_l = jnp.float32(1.0 / L)

  @plsc.parallel_loop(0, BW, unroll=4)
  def scale_body(i):
    acc_c[i, :] = acc_c[i, :] * inv_l
    acc_k[i, :] = acc_k[i, :] * inv_l

  rows = pl.ds(row0, BW)
  out_cps = [
      pltpu.make_async_copy(t_rows, out_hbm.at[rows, pl.ds(0, D_TITLE)], sem_o),
      pltpu.make_async_copy(acc_c, out_hbm.at[rows, pl.ds(32, D_CAST)], sem_o),
      pltpu.make_async_copy(acc_k, out_hbm.at[rows, pl.ds(48, D_KEY)], sem_o),
      pltpu.make_async_copy(m_rows, out_hbm.at[rows, pl.ds(64, D_MOOD)], sem_o),
  ]
  for cp in out_cps:
    cp.start()
  for cp in out_cps:
    cp.wait()


@jax.jit
def _run(title_ids, castT, keyT, mood_ids,
         title_table, cast_table, key_table, mood_table):
  mesh = plsc.VectorSubcoreMesh(core_axis_name="c", subcore_axis_name="s")

  def tail_of(tbl):
    t = lax.slice_in_dim(tbl, NFULL * CBLK, V, axis=0)
    return jnp.pad(t, ((0, TAIL_PAD - TAIL), (0, 0))).T

  cast_lin, key_lin, title_lin = pl.kernel(
      _detile_kernel,
      mesh=mesh,
      compiler_params=pltpu.CompilerParams(
          use_tc_tiling_on_sc=True, needs_layout_passes=False),
      out_type=(
          jax.ShapeDtypeStruct((V * D_CAST,), jnp.float32),
          jax.ShapeDtypeStruct((V * D_KEY,), jnp.float32),
          jax.ShapeDtypeStruct((V * D_TITLE,), jnp.float32),
      ),
      scratch_types=[
          pltpu.VMEM((16, CBLK), jnp.float32),
          pltpu.VMEM((16, CBLK), jnp.float32),
          pltpu.VMEM((CBLK * 16,), jnp.float32),
          pltpu.VMEM((CBLK * 16,), jnp.float32),
          pltpu.VMEM((32, CBLK), jnp.float32),
          pltpu.VMEM((32, CBLK), jnp.float32),
          pltpu.VMEM((CBLK * 32,), jnp.float32),
          pltpu.VMEM((CBLK * 32,), jnp.float32),
          pltpu.SemaphoreType.DMA,
          pltpu.SemaphoreType.DMA,
      ],
  )(cast_table.T, key_table.T, title_table.T,
    tail_of(cast_table), tail_of(key_table), tail_of(title_table))

  cast_u = cast_lin.reshape(V, D_CAST)
  key_u = key_lin.reshape(V, D_KEY)
  title_u = title_lin.reshape(V, D_TITLE)

  return pl.kernel(
      _lookup_kernel,
      mesh=mesh,
      compiler_params=pltpu.CompilerParams(use_tc_tiling_on_sc=False),
      out_type=jax.ShapeDtypeStruct((B, D_OUT), jnp.float32),
      scratch_types=[
          pltpu.VMEM((BW,), jnp.int32),
          pltpu.VMEM((BW,), jnp.int32),
          pltpu.VMEM((L, BW), jnp.int32),
          pltpu.VMEM((L, BW), jnp.int32),
          pltpu.VMEM((BW, D_TITLE), jnp.float32),
          pltpu.VMEM((BW, D_MOOD), jnp.float32),
          pltpu.VMEM((BW, D_CAST), jnp.float32),
          pltpu.VMEM((BW, D_KEY), jnp.float32),
          pltpu.SemaphoreType.DMA,
          pltpu.SemaphoreType.DMA,
          pltpu.SemaphoreType.DMA,
      ],
  )(title_ids, castT, keyT, mood_ids,
    title_u, cast_u, key_u, mood_table)


def kernel(title_ids, cast_ids, keyword_ids, mood_ids,
           title_table, cast_table, key_table, mood_table):
  return _run(title_ids, cast_ids.T, keyword_ids.T, mood_ids,
              title_table, cast_table, key_table, mood_table)
